# Initial kernel scaffold; baseline (speedup 1.0000x reference)
import jax
import jax.numpy as jnp
from jax.experimental import pallas as pl


def kernel(self_vector, neighbor_vectors, target_relation, neighbor_relations, relation_similarity, mask, att_a_self, att_a_nb, W_ih, b_ih, W_hh):
    raise NotImplementedError("write your pallas kernel here")


# speedup vs baseline: 14.0529x; 14.0529x over previous
import functools

import jax
import jax.numpy as jnp
from jax.experimental import pallas as pl
from jax.experimental.pallas import tpu as pltpu

N, NB, D, R = 10000, 32, 128, 16
BN = 400


def _block_kernel(self_ref, nbv_ref, tr_ref, nbr_ref, rs_ref, mask_ref,
                  a_self_ref, a_nb_ref, wih_ref, bih_ref, whh_ref, out_ref):
    sv = self_ref[...]
    nbv = nbv_ref[...]
    tr = tr_ref[...]
    nbr = nbr_ref[...]
    maskf = mask_ref[...]

    a_self = a_self_ref[...]
    a_nb = a_nb_ref[...]
    att_self = jnp.sum(sv * a_self, axis=-1, keepdims=True)
    att_nb_feat = jnp.sum(nbv * a_nb[None, :, :], axis=-1)
    att_feat = att_nb_feat + att_self + 1.0

    rel_rows = jnp.zeros((tr.shape[0], R), dtype=jnp.float32)
    for r in range(R):
        sel = (tr == r).astype(jnp.float32)
        rel_rows = rel_rows + sel * rs_ref[r:r + 1, :]
    att_rela = jnp.zeros_like(nbr, dtype=jnp.float32)
    for k in range(R):
        att_rela = jnp.where(nbr == k, rel_rows[:, k:k + 1], att_rela)

    att = jnp.where(att_feat >= 0, att_feat, 0.01 * att_feat) * att_rela
    neg = jnp.float32(-1e30)
    att = jnp.where(maskf > 0, att, neg)
    att = att - jnp.max(att, axis=-1, keepdims=True)
    e = jnp.exp(att) * maskf
    att = e / jnp.sum(e, axis=-1, keepdims=True)

    nb_vec = jnp.sum(att[:, :, None] * nbv, axis=1)

    gi = jnp.dot(nb_vec, wih_ref[...], preferred_element_type=jnp.float32)
    gi = gi + bih_ref[...]
    gh = jnp.dot(sv, whh_ref[...], preferred_element_type=jnp.float32)
    ri, zi, hi = gi[:, :D], gi[:, D:2 * D], gi[:, 2 * D:]
    rh, zh, hh = gh[:, :D], gh[:, D:2 * D], gh[:, 2 * D:]
    r = jax.nn.sigmoid(ri + rh)
    z = jax.nn.sigmoid(zi + zh)
    h = jnp.tanh(hi + hh * r)
    out_ref[...] = (1.0 - z) * sv + z * h


@jax.jit
def _run(self_vector, neighbor_vectors, tr2, nbr, rs, maskf,
         a_self, a_nb, wih_t, bih2, whh_t):
    grid = (N // BN,)
    const = lambda i: (0, 0)
    return pl.pallas_call(
        _block_kernel,
        grid=grid,
        in_specs=[
            pl.BlockSpec((BN, D), lambda i: (i, 0)),
            pl.BlockSpec((BN, NB, D), lambda i: (i, 0, 0)),
            pl.BlockSpec((BN, 1), lambda i: (i, 0)),
            pl.BlockSpec((BN, NB), lambda i: (i, 0)),
            pl.BlockSpec((R, R), const),
            pl.BlockSpec((BN, NB), lambda i: (i, 0)),
            pl.BlockSpec((1, D), const),
            pl.BlockSpec((1, D), const),
            pl.BlockSpec((D, 3 * D), const),
            pl.BlockSpec((1, 3 * D), const),
            pl.BlockSpec((D, 3 * D), const),
        ],
        out_specs=pl.BlockSpec((BN, D), lambda i: (i, 0)),
        out_shape=jax.ShapeDtypeStruct((N, D), jnp.float32),
        compiler_params=pltpu.CompilerParams(
            dimension_semantics=("arbitrary",),
        ),
    )(self_vector, neighbor_vectors, tr2, nbr, rs, maskf,
      a_self, a_nb, wih_t, bih2, whh_t)


def kernel(self_vector, neighbor_vectors, target_relation, neighbor_relations,
           relation_similarity, mask, att_a_self, att_a_nb, W_ih, b_ih, W_hh):
    tr2 = target_relation.astype(jnp.int32).reshape(N, 1)
    nbr = neighbor_relations.astype(jnp.int32)
    maskf = mask.astype(jnp.float32)
    a_self = att_a_self.reshape(1, D)
    a_nb = att_a_nb.reshape(1, D)
    wih_t = W_ih.T
    whh_t = W_hh.T
    bih2 = b_ih.reshape(1, 3 * D)
    return _run(self_vector, neighbor_vectors, tr2, nbr,
                relation_similarity, maskf, a_self, a_nb, wih_t, bih2, whh_t)
